# baseline (speedup 1.0000x reference)
import functools

import jax
import jax.numpy as jnp
from jax import lax
from jax.experimental import pallas as pl
from jax.experimental.pallas import tpu as pltpu
from jax.experimental.pallas import tpu_sc as plsc


_INV2PI = 0.15915493667125702
_TWOPI = 6.283185307179586
_CC = (0.999999443678766, -0.49999558165578417, 0.04166103279005172,
       -0.001386274731578642, 2.425319249599542e-05, -2.2193949944101022e-07)


def _sc_gather(table, fidx, tidx):
    E = fidx.shape[0]
    N, D = table.shape
    C = 128
    CH = E // C
    info = plsc.get_sparse_core_info()
    NC = info.num_cores
    NW = NC * info.num_subcores
    n_iter = -(-CH // NW)
    mesh = plsc.VectorSubcoreMesh(core_axis_name="c", subcore_axis_name="s")

    S = 2

    @functools.partial(
        pl.kernel,
        mesh=mesh,
        out_type=(jax.ShapeDtypeStruct((E, D), jnp.float32),
                  jax.ShapeDtypeStruct((E, D), jnp.float32)),
        scratch_types=[
            pltpu.VMEM((S * C,), jnp.int32),
            pltpu.VMEM((S * C,), jnp.int32),
            pltpu.VMEM((S * C, D), jnp.float32),
            pltpu.VMEM((S * C, D), jnp.float32),
            pltpu.SemaphoreType.DMA((S,)),
            pltpu.SemaphoreType.DMA((S,)),
            pltpu.SemaphoreType.DMA((S,)),
        ],
    )
    def k(table_h, fidx_h, tidx_h, fout_h, tout_h,
          fiv, tiv, gb, hb, sem_i, sem_g, sem_w):
        wid = lax.axis_index("s") * NC + lax.axis_index("c")

        def stage_in(g):
            r0 = (g * NW + wid) * C
            s = g % S
            pltpu.async_copy(fidx_h.at[pl.ds(r0, C)], fiv.at[pl.ds(s * C, C)], sem_i.at[s])
            pltpu.async_copy(tidx_h.at[pl.ds(r0, C)], tiv.at[pl.ds(s * C, C)], sem_i.at[s])

        def stage_gather(g):
            r0 = (g * NW + wid) * C
            s = g % S
            pltpu.make_async_copy(fidx_h.at[pl.ds(r0, C)], fiv.at[pl.ds(s * C, C)], sem_i.at[s]).wait()
            pltpu.make_async_copy(tidx_h.at[pl.ds(r0, C)], tiv.at[pl.ds(s * C, C)], sem_i.at[s]).wait()
            pltpu.async_copy(table_h.at[fiv.at[pl.ds(s * C, C)]], gb.at[pl.ds(s * C, C)], sem_g.at[s])
            pltpu.async_copy(table_h.at[tiv.at[pl.ds(s * C, C)]], hb.at[pl.ds(s * C, C)], sem_g.at[s])

        def stage_out(g):
            r0 = (g * NW + wid) * C
            s = g % S
            pltpu.make_async_copy(table_h.at[fiv.at[pl.ds(s * C, C)]], gb.at[pl.ds(s * C, C)], sem_g.at[s]).wait()
            pltpu.make_async_copy(table_h.at[tiv.at[pl.ds(s * C, C)]], hb.at[pl.ds(s * C, C)], sem_g.at[s]).wait()
            pltpu.async_copy(gb.at[pl.ds(s * C, C)], fout_h.at[pl.ds(r0, C)], sem_w.at[s])
            pltpu.async_copy(hb.at[pl.ds(s * C, C)], tout_h.at[pl.ds(r0, C)], sem_w.at[s])

        def stage_drain(g):
            r0 = (g * NW + wid) * C
            s = g % S
            pltpu.make_async_copy(gb.at[pl.ds(s * C, C)], fout_h.at[pl.ds(r0, C)], sem_w.at[s]).wait()
            pltpu.make_async_copy(hb.at[pl.ds(s * C, C)], tout_h.at[pl.ds(r0, C)], sem_w.at[s]).wait()

        def live(g):
            return jnp.logical_and(g >= 0, (g * NW + wid) < CH)

        def body(g, carry):
            @pl.when(live(g - 2))
            def _():
                stage_drain(g - 2)

            @pl.when(live(g))
            def _():
                stage_in(g)

            @pl.when(live(g - 1))
            def _():
                stage_out(g - 1)

            @pl.when(live(g))
            def _():
                stage_gather(g)

            return carry

        lax.fori_loop(0, n_iter + S, body, 0)

    return k(table, fidx, tidx)


def _assemble_tc(from2, to2, edge2t, t2, time_w, time_b, out_w, E, off, prev):
    EH, D = from2.shape
    DE = edge2t.shape[0]
    DT = time_w.shape[0]
    B = 1280
    G = EH // B
    G0 = off // B
    assert EH == G * B and off == G0 * B

    def body(f_ref, g_ref, e_ref, t_ref, w_ref, b_ref, *o_ref):
        x = w_ref[...] * t_ref[...] + b_ref[...]
        r = x - jnp.floor(x * _INV2PI + 0.5) * _TWOPI
        u = r * r
        p = jnp.float32(_CC[5])
        for c in (_CC[4], _CC[3], _CC[2], _CC[1], _CC[0]):
            p = p * u + c
        o_ref[-1][...] = jnp.concatenate(
            [f_ref[...].T, e_ref[...], g_ref[...].T, p], axis=0)

    in_specs = [
        pl.BlockSpec((B, D), lambda i: (i, 0)),
        pl.BlockSpec((B, D), lambda i: (i, 0)),
        pl.BlockSpec((DE, B), lambda i: (0, i + G0)),
        pl.BlockSpec((1, B), lambda i: (0, i + G0)),
        pl.BlockSpec((DT, 1), lambda i: (0, 0)),
        pl.BlockSpec((DT, 1), lambda i: (0, 0)),
    ]
    args = [from2, to2, edge2t, t2, time_w.reshape(DT, 1), time_b.reshape(DT, 1)]
    aliases = {}
    if prev is not None:
        in_specs.append(pl.BlockSpec(memory_space=pl.ANY))
        args.append(prev)
        aliases = {6: 0}
    return pl.pallas_call(
        body,
        grid=(G,),
        in_specs=in_specs,
        out_specs=pl.BlockSpec((out_w, B), lambda i: (0, i + G0)),
        out_shape=jax.ShapeDtypeStruct((out_w, E), jnp.float32),
        input_output_aliases=aliases,
    )(*args)


def kernel(update_node_emb, edge_emb, from_idx, to_idx, t, time_w, time_b):
    N, D = update_node_emb.shape
    E, DE = edge_emb.shape
    DT = time_w.shape[0]
    out_w = D + DE + D + DT

    fidx = from_idx.astype(jnp.int32)
    tidx = to_idx.astype(jnp.int32)
    E2 = E // 2
    fa, ta = _sc_gather(update_node_emb, fidx[:E2], tidx[:E2])
    fb, tb = _sc_gather(update_node_emb, fidx[E2:], tidx[E2:])
    edge2t = edge_emb.T
    t2 = t.reshape(1, E)
    o1 = _assemble_tc(fa, ta, edge2t, t2, time_w, time_b, out_w, E, 0, None)
    o2 = _assemble_tc(fb, tb, edge2t, t2, time_w, time_b, out_w, E, E2, o1)
    return o2.T


# speedup vs baseline: 1.0757x; 1.0757x over previous
import functools

import jax
import jax.numpy as jnp
from jax import lax
from jax.experimental import pallas as pl
from jax.experimental.pallas import tpu as pltpu
from jax.experimental.pallas import tpu_sc as plsc


_INV2PI = 0.15915493667125702
_TWOPI = 6.283185307179586
_CC = (0.999999443678766, -0.49999558165578417, 0.04166103279005172,
       -0.001386274731578642, 2.425319249599542e-05, -2.2193949944101022e-07)


def _sc_gather(table, fidx, tidx):
    E = fidx.shape[0]
    N, D = table.shape
    C = 128
    CH = E // C
    info = plsc.get_sparse_core_info()
    NC = info.num_cores
    NW = NC * info.num_subcores
    n_iter = -(-CH // NW)
    mesh = plsc.VectorSubcoreMesh(core_axis_name="c", subcore_axis_name="s")

    S = 2

    @functools.partial(
        pl.kernel,
        mesh=mesh,
        out_type=(jax.ShapeDtypeStruct((E, D), jnp.float32),
                  jax.ShapeDtypeStruct((E, D), jnp.float32)),
        scratch_types=[
            pltpu.VMEM((S * C,), jnp.int32),
            pltpu.VMEM((S * C,), jnp.int32),
            pltpu.VMEM((S * C, D), jnp.float32),
            pltpu.VMEM((S * C, D), jnp.float32),
            pltpu.SemaphoreType.DMA((S,)),
            pltpu.SemaphoreType.DMA((S,)),
            pltpu.SemaphoreType.DMA((S,)),
        ],
    )
    def k(table_h, fidx_h, tidx_h, fout_h, tout_h,
          fiv, tiv, gb, hb, sem_i, sem_g, sem_w):
        wid = lax.axis_index("s") * NC + lax.axis_index("c")

        def stage_in(g):
            r0 = (g * NW + wid) * C
            s = g % S
            pltpu.async_copy(fidx_h.at[pl.ds(r0, C)], fiv.at[pl.ds(s * C, C)], sem_i.at[s])
            pltpu.async_copy(tidx_h.at[pl.ds(r0, C)], tiv.at[pl.ds(s * C, C)], sem_i.at[s])

        def stage_gather(g):
            r0 = (g * NW + wid) * C
            s = g % S
            pltpu.make_async_copy(fidx_h.at[pl.ds(r0, C)], fiv.at[pl.ds(s * C, C)], sem_i.at[s]).wait()
            pltpu.make_async_copy(tidx_h.at[pl.ds(r0, C)], tiv.at[pl.ds(s * C, C)], sem_i.at[s]).wait()
            pltpu.async_copy(table_h.at[fiv.at[pl.ds(s * C, C)]], gb.at[pl.ds(s * C, C)], sem_g.at[s])
            pltpu.async_copy(table_h.at[tiv.at[pl.ds(s * C, C)]], hb.at[pl.ds(s * C, C)], sem_g.at[s])

        def stage_out(g):
            r0 = (g * NW + wid) * C
            s = g % S
            pltpu.make_async_copy(table_h.at[fiv.at[pl.ds(s * C, C)]], gb.at[pl.ds(s * C, C)], sem_g.at[s]).wait()
            pltpu.make_async_copy(table_h.at[tiv.at[pl.ds(s * C, C)]], hb.at[pl.ds(s * C, C)], sem_g.at[s]).wait()
            pltpu.async_copy(gb.at[pl.ds(s * C, C)], fout_h.at[pl.ds(r0, C)], sem_w.at[s])
            pltpu.async_copy(hb.at[pl.ds(s * C, C)], tout_h.at[pl.ds(r0, C)], sem_w.at[s])

        def stage_drain(g):
            r0 = (g * NW + wid) * C
            s = g % S
            pltpu.make_async_copy(gb.at[pl.ds(s * C, C)], fout_h.at[pl.ds(r0, C)], sem_w.at[s]).wait()
            pltpu.make_async_copy(hb.at[pl.ds(s * C, C)], tout_h.at[pl.ds(r0, C)], sem_w.at[s]).wait()

        def live(g):
            return jnp.logical_and(g >= 0, (g * NW + wid) < CH)

        def body(g, carry):
            @pl.when(live(g - 2))
            def _():
                stage_drain(g - 2)

            @pl.when(live(g))
            def _():
                stage_in(g)

            @pl.when(live(g - 1))
            def _():
                stage_out(g - 1)

            @pl.when(live(g))
            def _():
                stage_gather(g)

            return carry

        lax.fori_loop(0, n_iter + S, body, 0)

    return k(table, fidx, tidx)


def _assemble_tc(from2, to2, edge2, t, time_w, time_b, out_w):
    E, D = from2.shape
    DE = edge2.shape[-1]
    DT = time_w.shape[0]
    B = 2560
    G = E // B
    assert E == G * B

    def body(f_ref, g_ref, e_ref, t_ref, w_ref, b_ref, o_ref):
        x = w_ref[...] * t_ref[...] + b_ref[...]
        r = x - jnp.floor(x * _INV2PI + 0.5) * _TWOPI
        u = r * r
        p = jnp.float32(_CC[5])
        for c in (_CC[4], _CC[3], _CC[2], _CC[1], _CC[0]):
            p = p * u + c
        o_ref[...] = jnp.concatenate(
            [f_ref[...].T, e_ref[...], g_ref[...].T, p], axis=0)

    out_t = pl.pallas_call(
        body,
        grid=(G,),
        in_specs=[
            pl.BlockSpec((B, D), lambda i: (i, 0)),
            pl.BlockSpec((B, D), lambda i: (i, 0)),
            pl.BlockSpec((DE, B), lambda i: (0, i)),
            pl.BlockSpec((1, B), lambda i: (0, i)),
            pl.BlockSpec((DT, 1), lambda i: (0, 0)),
            pl.BlockSpec((DT, 1), lambda i: (0, 0)),
        ],
        out_specs=pl.BlockSpec((out_w, B), lambda i: (0, i)),
        out_shape=jax.ShapeDtypeStruct((out_w, E), jnp.float32),
    )(from2, to2, edge2.T, t.reshape(1, E),
      time_w.reshape(DT, 1), time_b.reshape(DT, 1))
    return out_t.T


def kernel(update_node_emb, edge_emb, from_idx, to_idx, t, time_w, time_b):
    N, D = update_node_emb.shape
    E, DE = edge_emb.shape
    DT = time_w.shape[0]
    out_w = D + DE + D + DT

    from2, to2 = _sc_gather(update_node_emb, from_idx.astype(jnp.int32),
                            to_idx.astype(jnp.int32))
    return _assemble_tc(from2, to2, edge_emb, t, time_w, time_b, out_w)
